# baseline (device time: 111908 ns/iter reference)
import jax
import jax.numpy as jnp
from jax import lax
from jax.experimental import pallas as pl
from jax.experimental.pallas import tpu as pltpu

N_DEV = 4
EPS = 1e-5


def kernel(x, Wp):
    B, Hs, W, C = x.shape
    Cout = Wp.shape[1]
    n_total = float(Hs * N_DEV * W)

    HBLK1 = 32
    NH1 = Hs // HBLK1
    HBLK2 = 16
    NH2 = Hs // HBLK2

    def body(x_ref, wp_ref, out_ref,
             xbf_ref, acc_ref, sc_ref, comm_ref, send_sems, recv_sems):
        h = pl.program_id(0)

        @pl.when(h < NH1)
        def _():
            xb = x_ref[...]
            xbf_ref[:, pl.ds(h * HBLK1, HBLK1)] = xb.astype(jnp.bfloat16)
            s = jnp.sum(xb, axis=(1, 2))
            ss = jnp.sum(xb * xb, axis=(1, 2))

            @pl.when(h == 0)
            def _():
                acc_ref[0] = s
                acc_ref[1] = ss

            @pl.when(h > 0)
            def _():
                acc_ref[0] += s
                acc_ref[1] += ss

        @pl.when(h == NH1 - 1)
        def _():
            my = lax.axis_index("i")

            barrier = pltpu.get_barrier_semaphore()
            for off in (1, 2, 3):
                pl.semaphore_signal(
                    barrier, inc=1,
                    device_id=((my + off) % N_DEV,),
                    device_id_type=pl.DeviceIdType.MESH,
                )
            pl.semaphore_wait(barrier, N_DEV - 1)

            comm_ref[my] = acc_ref[...]

            sends = []
            for off in (1, 2, 3):
                tgt = (my + off) % N_DEV
                rdma = pltpu.make_async_remote_copy(
                    src_ref=comm_ref.at[my],
                    dst_ref=comm_ref.at[my],
                    send_sem=send_sems.at[off - 1],
                    recv_sem=recv_sems.at[my],
                    device_id=(tgt,),
                    device_id_type=pl.DeviceIdType.MESH,
                )
                rdma.start()
                sends.append(rdma)

            for off in (1, 2, 3):
                src = (my - off) % N_DEV
                recv = pltpu.make_async_remote_copy(
                    src_ref=comm_ref.at[src],
                    dst_ref=comm_ref.at[src],
                    send_sem=send_sems.at[off - 1],
                    recv_sem=recv_sems.at[src],
                    device_id=(src,),
                    device_id_type=pl.DeviceIdType.MESH,
                )
                recv.wait_recv()
            for rdma in sends:
                rdma.wait_send()

            tot = comm_ref[0] + comm_ref[1] + comm_ref[2] + comm_ref[3]
            mean = tot[0] * (1.0 / n_total)
            ex2 = tot[1] * (1.0 / n_total)
            var = ex2 - mean * mean
            rstd = lax.rsqrt(var + EPS)
            sc_ref[0] = rstd
            sc_ref[1] = -mean * rstd

        @pl.when(h >= NH1)
        def _():
            k = h - NH1
            xb = xbf_ref[:, pl.ds(k * HBLK2, HBLK2)]
            scale = sc_ref[0].astype(jnp.bfloat16)[:, None, None, :]
            shift = sc_ref[1].astype(jnp.bfloat16)[:, None, None, :]
            hn = xb * scale + shift
            a = hn * jax.nn.sigmoid(hn)
            ab = a.reshape(B * HBLK2 * W, C)
            o = jnp.dot(ab, wp_ref[...].astype(jnp.bfloat16),
                        preferred_element_type=jnp.float32)
            out_ref[...] = o.astype(jnp.bfloat16).reshape(B, HBLK2, W, Cout)

    grid = NH1 + NH2
    out = pl.pallas_call(
        body,
        grid=(grid,),
        in_specs=[
            pl.BlockSpec((B, HBLK1, W, C),
                         lambda h: (0, jnp.minimum(h, NH1 - 1), 0, 0)),
            pl.BlockSpec((C, Cout), lambda h: (0, 0)),
        ],
        out_specs=pl.BlockSpec((B, HBLK2, W, Cout),
                               lambda h: (0, jnp.maximum(h - NH1, 0), 0, 0)),
        out_shape=jax.ShapeDtypeStruct((B, Hs, W, Cout), jnp.bfloat16),
        scratch_shapes=[
            pltpu.VMEM((B, Hs, W, C), jnp.bfloat16),
            pltpu.VMEM((2, B, C), jnp.float32),
            pltpu.VMEM((2, B, C), jnp.float32),
            pltpu.VMEM((N_DEV, 2, B, C), jnp.float32),
            pltpu.SemaphoreType.DMA((3,)),
            pltpu.SemaphoreType.DMA((N_DEV,)),
        ],
        compiler_params=pltpu.CompilerParams(
            collective_id=0,
            dimension_semantics=("arbitrary",),
            vmem_limit_bytes=64 * 1024 * 1024,
        ),
    )(x, Wp)
    return out


# device time: 72568 ns/iter; 1.5421x vs baseline; 1.5421x over previous
import jax
import jax.numpy as jnp
from jax import lax
from jax.experimental import pallas as pl
from jax.experimental.pallas import tpu as pltpu

N_DEV = 4
EPS = 1e-5


def kernel(x, Wp):
    B, Hs, W, C = x.shape
    Cout = Wp.shape[1]
    n_total = float(Hs * N_DEV * W)

    HBLK_A = 32
    NH_A = Hs // HBLK_A
    HBLK_B = 32
    NH_B = Hs // HBLK_B

    def stats_body(x_ref, stats_ref, acc_ref, comm_ref, send_sems, recv_sems):
        h = pl.program_id(0)
        xb = x_ref[...]
        s = jnp.sum(xb, axis=(1, 2))
        ss = jnp.sum(xb * xb, axis=(1, 2))

        @pl.when(h == 0)
        def _():
            acc_ref[0] = s
            acc_ref[1] = ss

        @pl.when(h > 0)
        def _():
            acc_ref[0] += s
            acc_ref[1] += ss

        @pl.when(h == NH_A - 1)
        def _():
            my = lax.axis_index("i")

            barrier = pltpu.get_barrier_semaphore()
            for off in (1, 2, 3):
                pl.semaphore_signal(
                    barrier, inc=1,
                    device_id=((my + off) % N_DEV,),
                    device_id_type=pl.DeviceIdType.MESH,
                )
            pl.semaphore_wait(barrier, N_DEV - 1)

            comm_ref[my] = acc_ref[...]

            sends = []
            for off in (1, 2, 3):
                tgt = (my + off) % N_DEV
                rdma = pltpu.make_async_remote_copy(
                    src_ref=comm_ref.at[my],
                    dst_ref=comm_ref.at[my],
                    send_sem=send_sems.at[off - 1],
                    recv_sem=recv_sems.at[my],
                    device_id=(tgt,),
                    device_id_type=pl.DeviceIdType.MESH,
                )
                rdma.start()
                sends.append(rdma)

            for off in (1, 2, 3):
                src = (my - off) % N_DEV
                recv = pltpu.make_async_remote_copy(
                    src_ref=comm_ref.at[src],
                    dst_ref=comm_ref.at[src],
                    send_sem=send_sems.at[off - 1],
                    recv_sem=recv_sems.at[src],
                    device_id=(src,),
                    device_id_type=pl.DeviceIdType.MESH,
                )
                recv.wait_recv()
            for rdma in sends:
                rdma.wait_send()

            tot = comm_ref[0] + comm_ref[1] + comm_ref[2] + comm_ref[3]
            mean = tot[0] * (1.0 / n_total)
            ex2 = tot[1] * (1.0 / n_total)
            var = ex2 - mean * mean
            rstd = lax.rsqrt(var + EPS)
            stats_ref[0] = rstd
            stats_ref[1] = -mean * rstd

    stats = pl.pallas_call(
        stats_body,
        grid=(NH_A,),
        in_specs=[
            pl.BlockSpec((B, HBLK_A, W, C), lambda h: (0, h, 0, 0)),
        ],
        out_specs=pl.BlockSpec((2, B, C), lambda h: (0, 0, 0)),
        out_shape=jax.ShapeDtypeStruct((2, B, C), jnp.float32),
        scratch_shapes=[
            pltpu.VMEM((2, B, C), jnp.float32),
            pltpu.VMEM((N_DEV, 2, B, C), jnp.float32),
            pltpu.SemaphoreType.DMA((3,)),
            pltpu.SemaphoreType.DMA((N_DEV,)),
        ],
        compiler_params=pltpu.CompilerParams(
            collective_id=0,
            dimension_semantics=("arbitrary",),
        ),
    )(x)

    def apply_body(x_ref, stats_ref, wp_ref, out_ref):
        xb = x_ref[...].astype(jnp.bfloat16)
        scale = stats_ref[0].astype(jnp.bfloat16)[:, None, None, :]
        shift = stats_ref[1].astype(jnp.bfloat16)[:, None, None, :]
        hn = xb * scale + shift
        a = (0.5 * hn) * (jnp.tanh(0.5 * hn) + 1.0)
        ab = a.reshape(B * HBLK_B * W, C)
        wb = wp_ref[...].astype(jnp.bfloat16)
        o = jnp.dot(ab, wb, preferred_element_type=jnp.float32)
        out_ref[...] = o.astype(jnp.bfloat16).reshape(B, HBLK_B, W, Cout)

    out = pl.pallas_call(
        apply_body,
        grid=(NH_B,),
        in_specs=[
            pl.BlockSpec((B, HBLK_B, W, C), lambda h: (0, h, 0, 0)),
            pl.BlockSpec((2, B, C), lambda h: (0, 0, 0)),
            pl.BlockSpec((C, Cout), lambda h: (0, 0)),
        ],
        out_specs=pl.BlockSpec((B, HBLK_B, W, Cout), lambda h: (0, h, 0, 0)),
        out_shape=jax.ShapeDtypeStruct((B, Hs, W, Cout), jnp.bfloat16),
        compiler_params=pltpu.CompilerParams(
            dimension_semantics=("arbitrary",),
            vmem_limit_bytes=64 * 1024 * 1024,
        ),
    )(x, stats, Wp)
    return out
